# baseline (speedup 1.0000x reference)
import functools

import jax
import jax.numpy as jnp
from jax import lax
from jax.experimental import pallas as pl
from jax.experimental.pallas import tpu as pltpu

_NUM_EXPERTS = 64
_TOP_K = 8
_N_GROUP = 8
_TOPK_GROUP = 4
_EPG = _NUM_EXPERTS // _N_GROUP
_SCALE = 2.5
_BLOCK_T = 1024
_SPLIT = 4


def _route(scores, s_r, T):
    neg = jnp.float32(-jnp.inf)

    gs = []
    for g in range(_N_GROUP):
        band = s_r[g * _EPG:(g + 1) * _EPG, :]
        m1 = jnp.max(band, axis=0, keepdims=True)
        m2 = jnp.max(jnp.where(band == m1, neg, band),
                     axis=0, keepdims=True)
        gs.append(m1 + m2)
    gstack = jnp.concatenate(gs, axis=0)

    rank = jnp.zeros((_N_GROUP, T), dtype=jnp.int32)
    for r in range(1, _N_GROUP):
        rot = jnp.roll(gstack, -r, axis=0)
        beats = rot > gstack
        tie_rows = (lax.broadcasted_iota(jnp.int32, (_N_GROUP, T), 0)
                    >= _N_GROUP - r)
        beats = beats | ((rot == gstack) & tie_rows)
        rank = rank + beats.astype(jnp.int32)
    selg = rank < _TOPK_GROUP

    bands = []
    for g in range(_N_GROUP):
        keep = jnp.broadcast_to(selg[g:g + 1, :], (_EPG, T))
        bands.append(jnp.where(keep, s_r[g * _EPG:(g + 1) * _EPG, :], neg))
    masked = jnp.concatenate(bands, axis=0)

    sub64f = lax.broadcasted_iota(
        jnp.int32, (_NUM_EXPERTS, T), 0).astype(jnp.float32)
    combo = sub64f + scores
    idx_rows = []
    val_rows = []
    cur = masked
    for k in range(_TOP_K):
        m = jnp.max(cur, axis=0, keepdims=True)
        hit = cur == m
        c = jnp.max(jnp.where(hit, combo, neg), axis=0, keepdims=True)
        idxf = jnp.floor(c)
        idx_rows.append(idxf.astype(jnp.int32))
        val_rows.append(c - idxf)
        if k + 1 < _TOP_K:
            cur = jnp.where(hit, neg, cur)

    idx_t = jnp.concatenate(idx_rows, axis=0)
    val_t = jnp.concatenate(val_rows, axis=0)
    denom = jnp.sum(val_t, axis=0, keepdims=True) + 1e-20
    return idx_t, val_t / denom * _SCALE


def _gate_body(x_ref, w_ref, bias_ref, logits_ref, idx_ref, wout_ref):
    x = x_ref[...]
    logits_t = lax.dot_general(w_ref[...], x, (((1,), (1,)), ((), ())),
                               preferred_element_type=jnp.float32)
    logits_ref[...] = logits_t.T

    scores = 1.0 / (1.0 + jnp.exp(-logits_t))
    s_r = scores + bias_ref[...]

    T = x.shape[0]
    th = T // _SPLIT
    for h in range(_SPLIT):
        sl = slice(h * th, (h + 1) * th)
        idx_t, w_t = _route(scores[:, sl], s_r[:, sl], th)
        idx_ref[:, sl] = idx_t
        wout_ref[:, sl] = w_t


@functools.partial(jax.jit, static_argnames=())
def kernel(hidden_states, gate_weight, expert_bias):
    n_tokens, hidden = hidden_states.shape
    bias = expert_bias.reshape(_NUM_EXPERTS, 1)
    grid = (n_tokens // _BLOCK_T,)
    out_shapes = (
        jax.ShapeDtypeStruct((n_tokens, _NUM_EXPERTS), jnp.float32),
        jax.ShapeDtypeStruct((_TOP_K, n_tokens), jnp.int32),
        jax.ShapeDtypeStruct((_TOP_K, n_tokens), jnp.float32),
    )
    logits, idx_t, w_t = pl.pallas_call(
        _gate_body,
        grid=grid,
        in_specs=[
            pl.BlockSpec((_BLOCK_T, hidden), lambda i: (i, 0)),
            pl.BlockSpec((_NUM_EXPERTS, hidden), lambda i: (0, 0)),
            pl.BlockSpec((_NUM_EXPERTS, 1), lambda i: (0, 0)),
        ],
        out_specs=(
            pl.BlockSpec((_BLOCK_T, _NUM_EXPERTS), lambda i: (i, 0)),
            pl.BlockSpec((_TOP_K, _BLOCK_T), lambda i: (0, i)),
            pl.BlockSpec((_TOP_K, _BLOCK_T), lambda i: (0, i)),
        ),
        out_shape=out_shapes,
        compiler_params=pltpu.CompilerParams(
            dimension_semantics=("parallel",),
        ),
    )(hidden_states, gate_weight, bias)
    return (idx_t.T, w_t.T, logits)


# speedup vs baseline: 1.0022x; 1.0022x over previous
import functools

import jax
import jax.numpy as jnp
from jax import lax
from jax.experimental import pallas as pl
from jax.experimental.pallas import tpu as pltpu

_NUM_EXPERTS = 64
_TOP_K = 8
_N_GROUP = 8
_TOPK_GROUP = 4
_EPG = _NUM_EXPERTS // _N_GROUP
_SCALE = 2.5
_BLOCK_T = 1024
_SPLIT = 2


def _route(scores, s_r, T):
    neg = jnp.float32(-jnp.inf)

    gs = []
    for g in range(_N_GROUP):
        band = s_r[g * _EPG:(g + 1) * _EPG, :]
        m1 = jnp.max(band, axis=0, keepdims=True)
        m2 = jnp.max(jnp.where(band == m1, neg, band),
                     axis=0, keepdims=True)
        gs.append(m1 + m2)
    gstack = jnp.concatenate(gs, axis=0)

    rank = jnp.zeros((_N_GROUP, T), dtype=jnp.int32)
    for r in range(1, _N_GROUP):
        rot = jnp.roll(gstack, -r, axis=0)
        beats = rot > gstack
        tie_rows = (lax.broadcasted_iota(jnp.int32, (_N_GROUP, T), 0)
                    >= _N_GROUP - r)
        beats = beats | ((rot == gstack) & tie_rows)
        rank = rank + beats.astype(jnp.int32)
    selg = rank < _TOPK_GROUP

    bands = []
    for g in range(_N_GROUP):
        keep = jnp.broadcast_to(selg[g:g + 1, :], (_EPG, T))
        bands.append(jnp.where(keep, s_r[g * _EPG:(g + 1) * _EPG, :], neg))
    masked = jnp.concatenate(bands, axis=0)

    sub64f = lax.broadcasted_iota(
        jnp.int32, (_NUM_EXPERTS, T), 0).astype(jnp.float32)
    combo = sub64f + scores
    idx_rows = []
    val_rows = []
    cur = masked
    for k in range(_TOP_K):
        m = jnp.max(cur, axis=0, keepdims=True)
        hit = cur == m
        c = jnp.max(jnp.where(hit, combo, neg), axis=0, keepdims=True)
        idxf = jnp.floor(c)
        idx_rows.append(idxf.astype(jnp.int32))
        val_rows.append(c - idxf)
        if k + 1 < _TOP_K:
            cur = jnp.where(hit, neg, cur)

    idx_t = jnp.concatenate(idx_rows, axis=0)
    val_t = jnp.concatenate(val_rows, axis=0)
    denom = jnp.sum(val_t, axis=0, keepdims=True) + 1e-20
    return idx_t, val_t / denom * _SCALE


def _gate_body(x_ref, w_ref, bias_ref, logits_ref, idx_ref, wout_ref):
    x = x_ref[...]
    logits_t = lax.dot_general(w_ref[...], x, (((1,), (1,)), ((), ())),
                               preferred_element_type=jnp.float32)
    logits_ref[...] = logits_t.T

    scores = 1.0 / (1.0 + jnp.exp(-logits_t))
    s_r = scores + bias_ref[...]

    T = x.shape[0]
    th = T // _SPLIT
    for h in range(_SPLIT):
        sl = slice(h * th, (h + 1) * th)
        idx_t, w_t = _route(scores[:, sl], s_r[:, sl], th)
        idx_ref[:, sl] = idx_t
        wout_ref[:, sl] = w_t


@functools.partial(jax.jit, static_argnames=())
def kernel(hidden_states, gate_weight, expert_bias):
    n_tokens, hidden = hidden_states.shape
    bias = expert_bias.reshape(_NUM_EXPERTS, 1)
    grid = (n_tokens // _BLOCK_T,)
    out_shapes = (
        jax.ShapeDtypeStruct((n_tokens, _NUM_EXPERTS), jnp.float32),
        jax.ShapeDtypeStruct((_TOP_K, n_tokens), jnp.int32),
        jax.ShapeDtypeStruct((_TOP_K, n_tokens), jnp.float32),
    )
    logits, idx_t, w_t = pl.pallas_call(
        _gate_body,
        grid=grid,
        in_specs=[
            pl.BlockSpec((_BLOCK_T, hidden), lambda i: (i, 0)),
            pl.BlockSpec((_NUM_EXPERTS, hidden), lambda i: (0, 0)),
            pl.BlockSpec((_NUM_EXPERTS, 1), lambda i: (0, 0)),
        ],
        out_specs=(
            pl.BlockSpec((_BLOCK_T, _NUM_EXPERTS), lambda i: (i, 0)),
            pl.BlockSpec((_TOP_K, _BLOCK_T), lambda i: (0, i)),
            pl.BlockSpec((_TOP_K, _BLOCK_T), lambda i: (0, i)),
        ),
        out_shape=out_shapes,
        compiler_params=pltpu.CompilerParams(
            dimension_semantics=("parallel",),
        ),
    )(hidden_states, gate_weight, bias)
    return (idx_t.T, w_t.T, logits)
